# baseline (speedup 1.0000x reference)
import jax
import jax.numpy as jnp
from jax import lax
from jax.experimental import pallas as pl

N = 400
B = 4
D = 128
EPS = 1e-5
_F32 = jnp.float32

_ROW_VEC = 0
_ROW_W2 = 16
_ROW_FC1 = 144
_ROW_FC2 = 272
_ROW_FC3 = 400
_P_ROWS = 464


def _ln(x, g, b):
    mu = jnp.mean(x, axis=-1, keepdims=True)
    var = jnp.mean((x - mu) ** 2, axis=-1, keepdims=True)
    return (x - mu) * lax.rsqrt(var + EPS) * g + b


def _dot(a, c):
    return jnp.dot(a, c, preferred_element_type=_F32)


def _tdot(a, c):
    return lax.dot_general(a, c, (((0,), (0,)), ((), ())),
                           preferred_element_type=_F32)


def _fused_kernel(sc_ref, W1_ref, P_ref, logits_ref, zp_ref):
    SS = sc_ref[...]
    Sb = [SS[i * N:(i + 1) * N, :] for i in range(B)]
    P = P_ref[...]

    vrow = lambda r, w=128: P[_ROW_VEC + r:_ROW_VEC + r + 1, :w]
    b1, b2 = vrow(0), vrow(1)
    lnE_g, lnE_b = vrow(2), vrow(3)
    fc1_b, ln1_g, ln1_b = vrow(4), vrow(5), vrow(6)
    fc2_b, ln2_g, ln2_b = vrow(7, 64), vrow(8, 64), vrow(9, 64)
    fc3_b = vrow(10, 4)
    W2 = P[_ROW_W2:_ROW_W2 + 128, :]
    fc1_W = P[_ROW_FC1:_ROW_FC1 + 128, :]
    fc2_W = P[_ROW_FC2:_ROW_FC2 + 128, :64]
    fc3_W = P[_ROW_FC3:_ROW_FC3 + 64, :4]

    ones = jnp.ones((N, 1), _F32)
    dinv = []
    for i in range(B):
        deg = _tdot(Sb[i], ones)
        dinv.append(jnp.where(deg > 0, lax.rsqrt(deg), 0.0))

    h_all = _dot(SS, W1_ref[...])
    x1 = []
    for i in range(B):
        h = h_all[i * N:(i + 1) * N, :]
        x1.append(jnp.maximum(
            _tdot(Sb[i], h * dinv[i]) * dinv[i] + b1, 0.0))

    h2_all = _dot(jnp.concatenate(x1, axis=0), W2)
    zrows = []
    for i in range(B):
        h = h2_all[i * N:(i + 1) * N, :]
        y = _tdot(Sb[i], h * dinv[i]) * dinv[i] + b2
        y = _ln(y, lnE_g, lnE_b)
        zrows.append(jnp.mean(y, axis=0, keepdims=True))

    z = jnp.concatenate(zrows, axis=0)
    zp_ref[...] = z
    hh = _dot(z, fc1_W) + fc1_b
    hh = jnp.maximum(_ln(hh, ln1_g, ln1_b), 0.0)
    hh = _dot(hh, fc2_W) + fc2_b
    hh = jnp.maximum(_ln(hh, ln2_g, ln2_b), 0.0)
    logits_ref[...] = _dot(hh, fc3_W) + fc3_b


def kernel(sc_matrix, W1, b1, W2, b2, lnE_g, lnE_b, fc1_W, fc1_b, ln1_g,
           ln1_b, fc2_W, fc2_b, ln2_g, ln2_b, fc3_W, fc3_b):
    padv = lambda v: jnp.pad(v, (0, 128 - v.shape[0])).reshape(1, 128)
    vec_block = jnp.concatenate(
        [padv(v) for v in (b1, b2, lnE_g, lnE_b, fc1_b, ln1_g, ln1_b,
                           fc2_b, ln2_g, ln2_b, fc3_b)]
        + [jnp.zeros((5, 128), _F32)], axis=0)
    packed = jnp.concatenate([
        vec_block,
        W2,
        fc1_W,
        jnp.pad(fc2_W, ((0, 0), (0, 64))),
        jnp.pad(fc3_W, ((0, 0), (0, 124))),
    ], axis=0)

    logits, zp = pl.pallas_call(
        _fused_kernel,
        out_shape=[
            jax.ShapeDtypeStruct((B, 4), _F32),
            jax.ShapeDtypeStruct((B, D), _F32),
        ],
    )(sc_matrix.reshape(B * N, N), W1, packed)
    return (logits, zp)


# speedup vs baseline: 1.6998x; 1.6998x over previous
import jax
import jax.numpy as jnp
from jax import lax
from jax.experimental import pallas as pl

N = 400
B = 4
D = 128
EPS = 1e-5
_F32 = jnp.float32


def _ln(x, g, b):
    mu = jnp.mean(x, axis=-1, keepdims=True)
    var = jnp.mean((x - mu) ** 2, axis=-1, keepdims=True)
    return (x - mu) * lax.rsqrt(var + EPS) * g + b


def _dot(a, c):
    return jnp.dot(a, c, preferred_element_type=_F32)


def _tdot(a, c):
    return lax.dot_general(a, c, (((0,), (0,)), ((), ())),
                           preferred_element_type=_F32)


def _fused_kernel(sc_ref, W1_ref, b1_ref, W2_ref, b2_ref, lnEg_ref, lnEb_ref,
                  fc1W_ref, fc1b_ref, ln1g_ref, ln1b_ref,
                  fc2W_ref, fc2b_ref, ln2g_ref, ln2b_ref,
                  fc3W_ref, fc3b_ref,
                  logits_ref, zp_ref):
    SS = sc_ref[...]
    Sb = [SS[i * N:(i + 1) * N, :] for i in range(B)]

    rb = lax.broadcasted_iota(jnp.int32, (B * N, B), 0) // N
    cb = lax.broadcasted_iota(jnp.int32, (B * N, B), 1)
    O = (rb == cb).astype(_F32)
    degs = _tdot(SS, O)
    dinv_all = jnp.where(degs > 0, lax.rsqrt(degs), 0.0)
    dinv = [dinv_all[:, i:i + 1] for i in range(B)]

    h_all = _dot(SS, W1_ref[...])
    x1 = []
    for i in range(B):
        h = h_all[i * N:(i + 1) * N, :]
        x1.append(jnp.maximum(
            _tdot(Sb[i], h * dinv[i]) * dinv[i] + b1_ref[...], 0.0))

    h2_all = _dot(jnp.concatenate(x1, axis=0), W2_ref[...])
    for i in range(B):
        h = h2_all[i * N:(i + 1) * N, :]
        y = _tdot(Sb[i], h * dinv[i]) * dinv[i] + b2_ref[...]
        y = _ln(y, lnEg_ref[...], lnEb_ref[...])
        zp_ref[pl.ds(i, 1), :] = jnp.mean(y, axis=0, keepdims=True)

    z = zp_ref[...]
    hh = _dot(z, fc1W_ref[...]) + fc1b_ref[...]
    hh = jnp.maximum(_ln(hh, ln1g_ref[...], ln1b_ref[...]), 0.0)
    hh = _dot(hh, fc2W_ref[...]) + fc2b_ref[...]
    hh = jnp.maximum(_ln(hh, ln2g_ref[...], ln2b_ref[...]), 0.0)
    logits_ref[...] = _dot(hh, fc3W_ref[...]) + fc3b_ref[...]


def kernel(sc_matrix, W1, b1, W2, b2, lnE_g, lnE_b, fc1_W, fc1_b, ln1_g,
           ln1_b, fc2_W, fc2_b, ln2_g, ln2_b, fc3_W, fc3_b):
    r2 = lambda v: v.reshape(1, -1)
    logits, zp = pl.pallas_call(
        _fused_kernel,
        out_shape=[
            jax.ShapeDtypeStruct((B, 4), _F32),
            jax.ShapeDtypeStruct((B, D), _F32),
        ],
    )(sc_matrix.reshape(B * N, N), W1, r2(b1), W2, r2(b2), r2(lnE_g),
      r2(lnE_b), fc1_W, r2(fc1_b), r2(ln1_g), r2(ln1_b),
      fc2_W, r2(fc2_b), r2(ln2_g), r2(ln2_b),
      fc3_W, r2(fc3_b))
    return (logits, zp)
